# baseline (speedup 1.0000x reference)
import jax
import jax.numpy as jnp
from jax.experimental import pallas as pl
from jax.experimental.pallas import tpu as pltpu

_NUM_SAMPLES = 16
_T = 128

_IDX = [(i * (_T - 1)) // (_NUM_SAMPLES - 1) for i in range(_NUM_SAMPLES)]


def _gather_frames(x_ref, o_ref, sems):
    copies = [
        pltpu.make_async_copy(x_ref.at[src], o_ref.at[dst], sems.at[dst])
        for dst, src in enumerate(_IDX)
    ]
    for c in copies:
        c.start()
    for c in copies:
        c.wait()


def kernel(x):
    return pl.pallas_call(
        _gather_frames,
        in_specs=[pl.BlockSpec(memory_space=pltpu.MemorySpace.HBM)],
        out_specs=pl.BlockSpec(memory_space=pltpu.MemorySpace.HBM),
        out_shape=jax.ShapeDtypeStruct((_NUM_SAMPLES,) + x.shape[1:], x.dtype),
        scratch_shapes=[pltpu.SemaphoreType.DMA((_NUM_SAMPLES,))],
    )(x)


# speedup vs baseline: 1.9912x; 1.9912x over previous
import jax
import jax.numpy as jnp
from jax import lax
from jax.experimental import pallas as pl
from jax.experimental.pallas import tpu as pltpu
from jax.experimental.pallas import tpu_sc as plsc

_NUM_SAMPLES = 16
_T = 128
_ROW = 3 * 224 * 224
_HALF = _ROW // 2
_NCHUNK = 8
_C = _HALF // _NCHUNK


def _sc_gather(x_hbm, out_hbm, buf, rsem, wsem):
    cid = lax.axis_index("c")
    sid = lax.axis_index("s")
    wid = sid * 2 + cid
    frame = wid // 2
    half = wid % 2
    src = (frame * (_T - 1)) // (_NUM_SAMPLES - 1)
    src_base = src * _ROW + half * _HALF
    dst_base = frame * _ROW + half * _HALF

    reads = []
    writes = []
    for k in range(_NCHUNK):
        reads.append(pltpu.make_async_copy(
            x_hbm.at[pl.ds(src_base + k * _C, _C)],
            buf.at[pl.ds(k * _C, _C)],
            rsem.at[k],
        ))
        writes.append(pltpu.make_async_copy(
            buf.at[pl.ds(k * _C, _C)],
            out_hbm.at[pl.ds(dst_base + k * _C, _C)],
            wsem.at[k],
        ))
    for r in reads:
        r.start()
    for k in range(_NCHUNK):
        reads[k].wait()
        writes[k].start()
    for w in writes:
        w.wait()


def kernel(x):
    x1 = x.reshape(_T * _ROW)
    mesh = plsc.VectorSubcoreMesh(core_axis_name="c", subcore_axis_name="s")
    out = pl.kernel(
        _sc_gather,
        mesh=mesh,
        out_type=jax.ShapeDtypeStruct((_NUM_SAMPLES * _ROW,), jnp.float32),
        scratch_types=[
            pltpu.VMEM((_HALF,), jnp.float32),
            pltpu.SemaphoreType.DMA((_NCHUNK,)),
            pltpu.SemaphoreType.DMA((_NCHUNK,)),
        ],
    )(x1)
    return out.reshape(_NUM_SAMPLES, 3, 224, 224)
